# Initial kernel scaffold; baseline (speedup 1.0000x reference)
import jax
import jax.numpy as jnp
from jax.experimental import pallas as pl


def kernel(x, anchors):
    raise NotImplementedError("write your pallas kernel here")


# speedup vs baseline: 4.4012x; 4.4012x over previous
import functools

import jax
import jax.numpy as jnp
from jax import lax
from jax.experimental import pallas as pl
from jax.experimental.pallas import tpu as pltpu
from jax.experimental.pallas import tpu_sc as plsc

_NC = 2
_NS = 16
_NW = _NC * _NS
_L = 16
_CHUNK = 6400


def kernel(x, anchors):
    b, t = x.shape
    n = b * t
    num_anchors = anchors.shape[0]
    idx = x.reshape(n).astype(jnp.int32)
    tab_flat = anchors.reshape(num_anchors * 4)

    per_w = n // _NW
    n_chunks = per_w // _CHUNK
    groups = _CHUNK // _L

    mesh = plsc.VectorSubcoreMesh(core_axis_name="c", subcore_axis_name="s")

    @functools.partial(
        pl.kernel,
        out_type=jax.ShapeDtypeStruct((n * 4,), jnp.float32),
        mesh=mesh,
        compiler_params=pltpu.CompilerParams(
            needs_layout_passes=False, use_tc_tiling_on_sc=False
        ),
        scratch_types=[
            pltpu.VMEM((num_anchors * 4,), jnp.float32),
            pltpu.VMEM((_CHUNK,), jnp.int32),
            pltpu.VMEM((_CHUNK * 4,), jnp.float32),
        ],
    )
    def _gather(idx_hbm, tab_hbm, out_hbm, tab_v, idx_v, out_v):
        wid = lax.axis_index("s") * _NC + lax.axis_index("c")
        base = wid * per_w
        pltpu.sync_copy(tab_hbm, tab_v)
        lane4 = lax.iota(jnp.int32, _L) * 4

        def do_chunk(k, _):
            off = base + k * _CHUNK
            pltpu.sync_copy(idx_hbm.at[pl.ds(off, _CHUNK)], idx_v)

            def do_group(g, _):
                iv4 = idx_v[pl.ds(g * _L, _L)] * 4
                rows4 = lane4 + g * (_L * 4)
                for c in range(4):
                    vals = plsc.load_gather(tab_v, [iv4 + c])
                    plsc.store_scatter(out_v, [rows4 + c], vals)
                return 0

            lax.fori_loop(0, groups, do_group, 0)
            pltpu.sync_copy(out_v, out_hbm.at[pl.ds(off * 4, _CHUNK * 4)])
            return 0

        lax.fori_loop(0, n_chunks, do_chunk, 0)

    out = _gather(idx, tab_flat)
    return out.reshape(b, t, 4)


# speedup vs baseline: 4.4353x; 1.0078x over previous
import functools

import jax
import jax.numpy as jnp
from jax import lax
from jax.experimental import pallas as pl
from jax.experimental.pallas import tpu as pltpu
from jax.experimental.pallas import tpu_sc as plsc

_NC = 2
_NS = 16
_NW = _NC * _NS
_L = 16
_ROWS_PER_W = 128
_CHUNK_ROWS = 64


def kernel(x, anchors):
    b, t = x.shape
    n = b * t
    num_anchors = anchors.shape[0]
    tab_flat = anchors.reshape(num_anchors * 4)

    n_chunks = _ROWS_PER_W // _CHUNK_ROWS
    full_groups = t // _L
    tail_c0 = t - _L

    mesh = plsc.VectorSubcoreMesh(core_axis_name="c", subcore_axis_name="s")

    @functools.partial(
        pl.kernel,
        out_type=jax.ShapeDtypeStruct((n * 4,), jnp.float32),
        mesh=mesh,
        compiler_params=pltpu.CompilerParams(needs_layout_passes=False),
        scratch_types=[
            pltpu.VMEM((num_anchors * 4,), jnp.float32),
            pltpu.VMEM((_CHUNK_ROWS, t), jnp.int32),
            pltpu.VMEM((_CHUNK_ROWS * t * 4,), jnp.float32),
        ],
    )
    def _gather(x_hbm, tab_hbm, out_hbm, tab_v, idx_v, out_v):
        wid = lax.axis_index("s") * _NC + lax.axis_index("c")
        row0 = wid * _ROWS_PER_W
        pltpu.sync_copy(tab_hbm, tab_v)
        lane4 = lax.iota(jnp.int32, _L) * 4

        def do_group(r, c0):
            iv4 = idx_v[r, pl.ds(c0, _L)] * 4
            pos = lane4 + ((r * t + c0) * 4)
            for c in range(4):
                vals = plsc.load_gather(tab_v, [iv4 + c])
                plsc.store_scatter(out_v, [pos + c], vals)

        def do_chunk(k, _):
            r_base = row0 + k * _CHUNK_ROWS
            pltpu.sync_copy(x_hbm.at[pl.ds(r_base, _CHUNK_ROWS)], idx_v)

            def do_row(r, _):
                def body(g, _):
                    do_group(r, g * _L)
                    return 0

                lax.fori_loop(0, full_groups, body, 0)
                do_group(r, tail_c0)
                return 0

            lax.fori_loop(0, _CHUNK_ROWS, do_row, 0)
            pltpu.sync_copy(out_v, out_hbm.at[pl.ds(r_base * t * 4, _CHUNK_ROWS * t * 4)])
            return 0

        lax.fori_loop(0, n_chunks, do_chunk, 0)

    out = _gather(x.astype(jnp.int32), tab_flat)
    return out.reshape(b, t, 4)


# speedup vs baseline: 39.0718x; 8.8093x over previous
import functools

import jax
import jax.numpy as jnp
from jax import lax
from jax.experimental import pallas as pl
from jax.experimental.pallas import tpu as pltpu
from jax.experimental.pallas import tpu_sc as plsc

_NC = 2
_NS = 16
_NW = _NC * _NS
_L = 16
_BBLK = 128
_TCHUNK = 40


def kernel(x, anchors):
    b, t = x.shape
    num_anchors = anchors.shape[0]
    xt = x.T.astype(jnp.int32)
    tab_flat = anchors.reshape(num_anchors * 4)

    kblk = b // (2 * _BBLK)
    n_chunks = t // _TCHUNK
    groups = _BBLK // _L

    mesh = plsc.VectorSubcoreMesh(core_axis_name="c", subcore_axis_name="s")

    @functools.partial(
        pl.kernel,
        out_type=jax.ShapeDtypeStruct((t, kblk, 8, 128), jnp.float32),
        mesh=mesh,
        compiler_params=pltpu.CompilerParams(needs_layout_passes=False),
        scratch_types=[
            pltpu.VMEM((num_anchors * 4,), jnp.float32),
            pltpu.VMEM((_TCHUNK, _BBLK), jnp.int32),
            pltpu.VMEM((_TCHUNK, 4, 128), jnp.float32),
        ],
    )
    def _gather(xt_hbm, tab_hbm, out_hbm, tab_v, idx_v, out_v):
        wid = lax.axis_index("s") * _NC + lax.axis_index("c")
        kk = wid // 2
        r0 = (wid % 2) * 4
        pltpu.sync_copy(tab_hbm, tab_v)

        def do_chunk(ch, _):
            t0 = ch * _TCHUNK
            pltpu.sync_copy(
                xt_hbm.at[pl.ds(t0, _TCHUNK), pl.ds(wid * _BBLK, _BBLK)], idx_v
            )

            def do_row(tr, _):
                def do_group(g, _):
                    iv4 = idx_v[tr, pl.ds(g * _L, _L)] * 4
                    for c in range(4):
                        out_v[tr, c, pl.ds(g * _L, _L)] = plsc.load_gather(
                            tab_v, [iv4 + c]
                        )
                    return 0

                lax.fori_loop(0, groups, do_group, 0)
                return 0

            lax.fori_loop(0, _TCHUNK, do_row, 0)
            pltpu.sync_copy(
                out_v, out_hbm.at[pl.ds(t0, _TCHUNK), kk, pl.ds(r0, 4)]
            )
            return 0

        lax.fori_loop(0, n_chunks, do_chunk, 0)

    out4d = _gather(xt, tab_flat)
    out = (
        out4d.reshape(t, kblk, 2, 4, 128)
        .transpose(1, 2, 4, 0, 3)
        .reshape(b, t, 4)
    )
    return out


# speedup vs baseline: 72.5419x; 1.8566x over previous
import functools

import jax
import jax.numpy as jnp
from jax import lax
from jax.experimental import pallas as pl
from jax.experimental.pallas import tpu as pltpu
from jax.experimental.pallas import tpu_sc as plsc

_NC = 2
_NS = 16
_NW = _NC * _NS
_L = 16
_BBLK = 128
_TCHUNK = 40


def kernel(x, anchors):
    b, t = x.shape
    num_anchors = anchors.shape[0]
    xt = x.T.astype(jnp.int32)
    tab_flat = anchors.reshape(num_anchors * 4)

    kblk = b // (2 * _BBLK)
    n_chunks = t // _TCHUNK
    groups = _BBLK // _L

    mesh = plsc.VectorSubcoreMesh(core_axis_name="c", subcore_axis_name="s")

    @functools.partial(
        pl.kernel,
        out_type=jax.ShapeDtypeStruct((t, kblk, 8, 128), jnp.float32),
        mesh=mesh,
        compiler_params=pltpu.CompilerParams(needs_layout_passes=False),
        scratch_types=[
            pltpu.VMEM((num_anchors * 4,), jnp.float32),
            pltpu.VMEM((_TCHUNK, _BBLK), jnp.int32),
            pltpu.VMEM((_TCHUNK, 4, 128), jnp.float32),
        ],
    )
    def _gather(xt_hbm, tab_hbm, out_hbm, tab_v, idx_v, out_v):
        wid = lax.axis_index("s") * _NC + lax.axis_index("c")
        kk = wid // 2
        r0 = (wid % 2) * 4
        pltpu.sync_copy(tab_hbm, tab_v)

        def do_chunk(ch, _):
            t0 = ch * _TCHUNK
            pltpu.sync_copy(
                xt_hbm.at[pl.ds(t0, _TCHUNK), pl.ds(wid * _BBLK, _BBLK)], idx_v
            )

            @plsc.parallel_loop(0, _TCHUNK * groups, unroll=8)
            def _(i):
                tr = i // groups
                g = i % groups
                iv4 = idx_v[tr, pl.ds(g * _L, _L)] * 4
                for c in range(4):
                    out_v[tr, c, pl.ds(g * _L, _L)] = plsc.load_gather(
                        tab_v, [iv4 + c]
                    )
            pltpu.sync_copy(
                out_v, out_hbm.at[pl.ds(t0, _TCHUNK), kk, pl.ds(r0, 4)]
            )
            return 0

        lax.fori_loop(0, n_chunks, do_chunk, 0)

    out4d = _gather(xt, tab_flat)
    out = (
        out4d.reshape(t, kblk, 2, 4, 128)
        .transpose(1, 2, 4, 0, 3)
        .reshape(b, t, 4)
    )
    return out


# speedup vs baseline: 87.8773x; 1.2114x over previous
import functools

import jax
import jax.numpy as jnp
from jax import lax
from jax.experimental import pallas as pl
from jax.experimental.pallas import tpu as pltpu
from jax.experimental.pallas import tpu_sc as plsc

_NC = 2
_NS = 16
_NW = _NC * _NS
_L = 16
_BBLK = 128
_TCHUNK = 40


def kernel(x, anchors):
    b, t = x.shape
    num_anchors = anchors.shape[0]
    xt = x.T.astype(jnp.int32)
    tab_flat = anchors.reshape(num_anchors * 4)

    kblk = b // (2 * _BBLK)
    n_chunks = t // _TCHUNK
    groups = _BBLK // _L

    mesh = plsc.VectorSubcoreMesh(core_axis_name="c", subcore_axis_name="s")

    @functools.partial(
        pl.kernel,
        out_type=jax.ShapeDtypeStruct((t, kblk, 8, 128), jnp.float32),
        mesh=mesh,
        compiler_params=pltpu.CompilerParams(needs_layout_passes=False),
        scratch_types=[
            pltpu.VMEM((num_anchors * 4,), jnp.float32),
            pltpu.VMEM((2, _TCHUNK, _BBLK), jnp.int32),
            pltpu.VMEM((2, _TCHUNK, 4, 128), jnp.float32),
            pltpu.SemaphoreType.DMA,
            pltpu.SemaphoreType.DMA,
            pltpu.SemaphoreType.DMA,
            pltpu.SemaphoreType.DMA,
        ],
    )
    def _gather(xt_hbm, tab_hbm, out_hbm, tab_v, idx_v, out_v, is0, is1, os0, os1):
        wid = lax.axis_index("s") * _NC + lax.axis_index("c")
        kk = wid // 2
        r0 = (wid % 2) * 4
        isems = (is0, is1)
        osems = (os0, os1)

        def start_in(ch):
            return pltpu.async_copy(
                xt_hbm.at[
                    pl.ds(ch * _TCHUNK, _TCHUNK), pl.ds(wid * _BBLK, _BBLK)
                ],
                idx_v.at[ch % 2],
                isems[ch % 2],
            )

        in_cp = [None] * n_chunks
        out_cp = [None] * n_chunks
        in_cp[0] = start_in(0)
        pltpu.sync_copy(tab_hbm, tab_v)
        for ch in range(n_chunks):
            s = ch % 2
            if ch + 1 < n_chunks:
                in_cp[ch + 1] = start_in(ch + 1)
            in_cp[ch].wait()
            if ch >= 2:
                out_cp[ch - 2].wait()

            @plsc.parallel_loop(0, _TCHUNK * groups, unroll=8)
            def _(i):
                tr = i // groups
                g = i % groups
                iv4 = idx_v[s, tr, pl.ds(g * _L, _L)] * 4
                for c in range(4):
                    out_v[s, tr, c, pl.ds(g * _L, _L)] = plsc.load_gather(
                        tab_v, [iv4 + c]
                    )

            out_cp[ch] = pltpu.async_copy(
                out_v.at[s],
                out_hbm.at[pl.ds(ch * _TCHUNK, _TCHUNK), kk, pl.ds(r0, 4)],
                osems[s],
            )
        out_cp[n_chunks - 2].wait()
        out_cp[n_chunks - 1].wait()

    out4d = _gather(xt, tab_flat)
    out = (
        out4d.reshape(t, kblk, 2, 4, 128)
        .transpose(1, 2, 4, 0, 3)
        .reshape(b, t, 4)
    )
    return out
